# baseline (speedup 1.0000x reference)
import functools

import jax
import jax.numpy as jnp
from jax import lax
from jax.experimental import pallas as pl
from jax.experimental.pallas import tpu as pltpu
from jax.experimental.pallas import tpu_sc as plsc

N = 10000
EMB = 128
E = 320000
G = 16
STEPS = 3

NW = 32
NPW = (N + NW - 1) // NW
NPAD = NW * NPW
CAP = NPW + 1
K = 128
NEG = float("-inf")


def _leaky(z):
    return jnp.where(z >= 0, z, 0.01 * z)


def _sc_agg_body(xm_hbm, src_hbm, rest_hbm, wme_hbm, starts_hbm,
                 out_hbm,
                 idx0_v, idx1_v, rest0_v, rest1_v, wme_v, starts_v,
                 rows0_v, rows1_v, agg_v, sem0, sem1):
    wid = lax.axis_index("s") * 2 + lax.axis_index("c")
    base = wid * NPW

    pltpu.sync_copy(wme_hbm, wme_v)
    pltpu.sync_copy(starts_hbm, starts_v)

    iot = lax.broadcasted_iota(jnp.int32, (16,), 0)
    s0 = starts_v[pl.ds(0, 16)]
    s1 = starts_v[pl.ds(16, 16)]
    s2 = starts_v[pl.ds(32, 16)]

    vals = ([s0[j] for j in range(16)] + [s1[j] for j in range(16)]
            + [s2[j] for j in range(16)])

    def _pick(pos):
        r = vals[0]
        for idx in range(1, 33):
            r = jnp.where(pos == idx, vals[idx], r)
        return r

    lo = _pick(wid)
    hi = _pick(wid + 1)

    def init_body(i, _):
        agg_v[pl.ds(i * 16, 16)] = jnp.full((16,), NEG, jnp.float32)
        return 0
    lax.fori_loop(0, CAP * EMB // 16, init_body, 0)

    lo_al = (lo // K) * K
    nchunks = (hi - lo_al + K - 1) // K

    bufs = ((idx0_v, rest0_v, rows0_v, sem0), (idx1_v, rest1_v, rows1_v, sem1))

    def _stage(ci, idx_v, rest_v, rows_v, sem):
        e0 = lo_al + ci * K
        pltpu.sync_copy(src_hbm.at[pl.ds(e0, K)], idx_v)
        pltpu.sync_copy(rest_hbm.at[pl.ds(e0 * 5, 5 * K)], rest_v)
        return pltpu.async_copy(xm_hbm.at[idx_v], rows_v, sem)

    def _compute(ci, rest_v, rows_v):
        e0 = lo_al + ci * K

        def group_body(gi, _):
            gb = gi * 16
            evec = e0 + gb + iot
            dvec = rest_v[pl.ds(gb, 16)].astype(jnp.int32)
            validv = (evec >= lo) & (evec < hi)
            rbv = jnp.where(validv, dvec - base, NPW) * EMB
            a0v = rest_v[pl.ds(K + gb, 16)]
            a1v = rest_v[pl.ds(2 * K + gb, 16)]
            a2v = rest_v[pl.ds(3 * K + gb, 16)]
            a3v = rest_v[pl.ds(4 * K + gb, 16)]
            for j in range(16):
                rb = rbv[j]
                a0, a1, a2, a3 = a0v[j], a1v[j], a2v[j], a3v[j]
                k = gb + j
                for f in range(EMB // 16):
                    g = rows_v[k, pl.ds(f * 16, 16)]
                    w0 = wme_v[pl.ds(0 * EMB + f * 16, 16)]
                    w1 = wme_v[pl.ds(1 * EMB + f * 16, 16)]
                    w2 = wme_v[pl.ds(2 * EMB + f * 16, 16)]
                    w3 = wme_v[pl.ds(3 * EMB + f * 16, 16)]
                    msg = g + a0 * w0 + a1 * w1 + a2 * w2 + a3 * w3
                    off = rb + f * 16
                    agg_v[pl.ds(off, 16)] = jnp.maximum(
                        agg_v[pl.ds(off, 16)], msg)
            return 0
        lax.fori_loop(0, K // 16, group_body, 0)

    @pl.when(nchunks > 0)
    def _prologue():
        _stage(0, *bufs[0])

    def pair_body(ci2, _):
        for half in range(2):
            ci = ci2 * 2 + half
            cur, nxt = bufs[half], bufs[1 - half]

            @pl.when(ci < nchunks)
            def _one():
                @pl.when(ci + 1 < nchunks)
                def _prefetch():
                    _stage(ci + 1, *nxt)
                pltpu.make_async_copy(xm_hbm.at[cur[0]], cur[2],
                                      cur[3]).wait()
                _compute(ci, cur[1], cur[2])
        return 0

    lax.fori_loop(0, (nchunks + 1) // 2, pair_body, 0)

    pltpu.sync_copy(agg_v.at[pl.ds(0, NPW * EMB)],
                    out_hbm.at[pl.ds(base * EMB, NPW * EMB)])


_sc_agg = pl.kernel(
    _sc_agg_body,
    out_type=jax.ShapeDtypeStruct((NPAD * EMB,), jnp.float32),
    mesh=plsc.VectorSubcoreMesh(core_axis_name="c", subcore_axis_name="s"),
    scratch_types=[
        pltpu.VMEM((K,), jnp.int32),
        pltpu.VMEM((K,), jnp.int32),
        pltpu.VMEM((5 * K,), jnp.float32),
        pltpu.VMEM((5 * K,), jnp.float32),
        pltpu.VMEM((4 * EMB,), jnp.float32),
        pltpu.VMEM((48,), jnp.int32),
        pltpu.VMEM((K, EMB), jnp.float32),
        pltpu.VMEM((K, EMB), jnp.float32),
        pltpu.VMEM((CAP * EMB,), jnp.float32),
        pltpu.SemaphoreType.DMA,
        pltpu.SemaphoreType.DMA,
    ],
)


def _mm_body(x_ref, w_ref, o_ref):
    o_ref[...] = jnp.dot(x_ref[...], w_ref[...],
                         preferred_element_type=jnp.float32)


_mm = pl.pallas_call(
    _mm_body, out_shape=jax.ShapeDtypeStruct((N, EMB), jnp.float32))


def _tc_step_body(has_next, bi_c_ref, bi_r_ref, x_ref, xg_ref, raw_ref,
                  bm_ref, wax_ref, wag_ref, waa_ref, ba_ref, wgm_ref,
                  bgm_ref, wgf_ref, bgf_ref, wtp_ref, wtg_ref, bt_ref,
                  wmn_ref, *out_refs):
    x = x_ref[...]
    xg = xg_ref[...]
    raw = raw_ref[...][:N]
    bi_c = bi_c_ref[...]
    bi_r = bi_r_ref[...]
    oh = bi_c == lax.broadcasted_iota(jnp.int32, (1, G), 1)
    ohf = oh.astype(jnp.float32)
    oht = (bi_r == lax.broadcasted_iota(jnp.int32, (G, 1), 0))
    ohtf = oht.astype(jnp.float32)

    agg = jnp.where(jnp.isfinite(raw), _leaky(raw + bm_ref[...]), 0.0)
    brd = jnp.dot(ohf, jnp.dot(xg, wag_ref[...],
                               preferred_element_type=jnp.float32),
                  preferred_element_type=jnp.float32)
    u = (jnp.dot(x, wax_ref[...], preferred_element_type=jnp.float32)
         + jnp.dot(agg, waa_ref[...], preferred_element_type=jnp.float32)
         + brd + ba_ref[...])
    x2 = _leaky(u) + x

    gl = jnp.sum(x2 * wgm_ref[...], axis=1, keepdims=True) + bgm_ref[...]
    gmax = jnp.max(jnp.where(oh, gl, NEG), axis=0, keepdims=True)
    gmax_b = jnp.max(jnp.where(oh, gmax, NEG), axis=1, keepdims=True)
    ge = jnp.exp(gl - gmax_b)
    gsum = jnp.sum(jnp.where(oh, ge, 0.0), axis=0, keepdims=True)
    gsum_b = jnp.max(jnp.where(oh, gsum, 0.0), axis=1, keepdims=True)
    alpha = ge / (gsum_b + 1e-16)
    feat = _leaky(jnp.dot(x2, wgf_ref[...],
                          preferred_element_type=jnp.float32) + bgf_ref[...])
    pooled = jnp.dot(ohtf, alpha * feat,
                     preferred_element_type=jnp.float32)
    xg2 = _leaky(jnp.dot(pooled, wtp_ref[...],
                         preferred_element_type=jnp.float32)
                 + jnp.dot(xg, wtg_ref[...],
                           preferred_element_type=jnp.float32)
                 + bt_ref[...]) + xg

    out_refs[0][...] = x2
    out_refs[1][...] = xg2
    if has_next:
        out_refs[2][...] = jnp.dot(x2, wmn_ref[...],
                                   preferred_element_type=jnp.float32)


_tc_step = pl.pallas_call(
    functools.partial(_tc_step_body, True),
    out_shape=[jax.ShapeDtypeStruct((N, EMB), jnp.float32),
               jax.ShapeDtypeStruct((G, EMB), jnp.float32),
               jax.ShapeDtypeStruct((N, EMB), jnp.float32)])


def kernel(x, step_idx, edge_attr, edge_index, batch_ind, num_graphs,
           Wm, bm, Wa, ba, Wgm, bgm, Wgf, bgf, Wt, bt):
    src = edge_index[0].astype(jnp.int32)
    dst = edge_index[1].astype(jnp.int32)
    perm = jnp.argsort(dst)
    src_s = src[perm]
    dst_s = dst[perm]
    rest = (jnp.concatenate([dst_s.astype(jnp.float32)[None],
                             edge_attr[perm].T], axis=0)
            .reshape(5, E // K, K).swapaxes(0, 1).reshape(-1))
    bounds = jnp.arange(33, dtype=jnp.int32) * NPW
    starts = jnp.searchsorted(dst_s, bounds).astype(jnp.int32)
    starts = jnp.pad(starts, (0, 15), constant_values=E)
    bi_c = batch_ind.astype(jnp.int32).reshape(N, 1)
    bi_r = batch_ind.astype(jnp.int32).reshape(1, N)

    xg0 = jnp.zeros((G, EMB), jnp.float32)
    xm0 = _mm(x, Wm[0][:EMB])
    wm_next = jnp.roll(Wm, -1, axis=0)[:, :EMB]

    def step(carry, ws):
        xc, xgc, xmc = carry
        wmi, wai, bai, bmi, wgmi, bgmi, wgfi, bgfi, wti, bti, wmn = ws
        wme = wmi[EMB:].reshape(-1)
        raw = _sc_agg(xmc, src_s, rest, wme, starts).reshape(NPAD, EMB)
        x2, xg2, xmn = _tc_step(
            bi_c, bi_r, xc, xgc, raw,
            bmi.reshape(1, EMB),
            wai[:EMB], wai[EMB:2 * EMB], wai[2 * EMB:],
            bai.reshape(1, EMB),
            wgmi.reshape(1, EMB),
            bgmi.reshape(1, 1),
            wgfi, bgfi.reshape(1, EMB),
            wti[:EMB], wti[EMB:],
            bti.reshape(1, EMB),
            wmn)
        return (x2, xg2, xmn), None

    (xf, xgf, _), _ = lax.scan(
        step, (x, xg0, xm0),
        (Wm, Wa, ba, bm, Wgm, bgm, Wgf, bgf, Wt, bt, wm_next))
    return (xf, xgf)


# speedup vs baseline: 2.3465x; 2.3465x over previous
import functools

import jax
import jax.numpy as jnp
from jax import lax
from jax.experimental import pallas as pl
from jax.experimental.pallas import tpu as pltpu
from jax.experimental.pallas import tpu_sc as plsc

N = 10000
EMB = 128
E = 320000
G = 16
STEPS = 3

NW = 32
NPW = (N + NW - 1) // NW
NPAD = NW * NPW
CAP = NPW + 1
K = 128
NEG = float("-inf")


def _leaky(z):
    return jnp.where(z >= 0, z, 0.01 * z)


def _sc_agg_body(xm_hbm, src_hbm, dstf_hbm, eam_hbm, starts_hbm,
                 out_hbm,
                 idx_v, dstf_v, eam_v, starts_v, rows_v, agg_v, sem):
    wid = lax.axis_index("s") * 2 + lax.axis_index("c")
    base = wid * NPW

    pltpu.sync_copy(starts_hbm, starts_v)

    iot = lax.broadcasted_iota(jnp.int32, (16,), 0)
    s0 = starts_v[pl.ds(0, 16)]
    s1 = starts_v[pl.ds(16, 16)]
    s2 = starts_v[pl.ds(32, 16)]

    vals = ([s0[j] for j in range(16)] + [s1[j] for j in range(16)]
            + [s2[j] for j in range(16)])

    def _pick(pos):
        r = vals[0]
        for idx in range(1, 33):
            r = jnp.where(pos == idx, vals[idx], r)
        return r

    lo = _pick(wid)
    hi = _pick(wid + 1)

    def init_body(i, _):
        agg_v[pl.ds(i * 16, 16)] = jnp.full((16,), NEG, jnp.float32)
        return 0
    lax.fori_loop(0, CAP * EMB // 16, init_body, 0)

    lo_al = (lo // K) * K
    nchunks = (hi - lo_al + K - 1) // K

    def chunk_body(ci, _):
        e0 = lo_al + ci * K
        pltpu.sync_copy(src_hbm.at[pl.ds(e0, K)], idx_v)
        pltpu.sync_copy(dstf_hbm.at[pl.ds(e0, K)], dstf_v)
        gat = pltpu.async_copy(xm_hbm.at[idx_v], rows_v, sem)
        pltpu.sync_copy(eam_hbm.at[pl.ds(e0, K)], eam_v)
        gat.wait()

        def group_body(gi, _):
            gb = gi * 16
            evec = e0 + gb + iot
            dvec = dstf_v[pl.ds(gb, 16)].astype(jnp.int32)
            validv = (evec >= lo) & (evec < hi)
            rbv = jnp.where(validv, dvec - base, NPW) * EMB
            for j in range(16):
                rb = rbv[j]
                k = gb + j
                for f in range(EMB // 16):
                    msg = (rows_v[k, pl.ds(f * 16, 16)]
                           + eam_v[k, pl.ds(f * 16, 16)])
                    off = rb + f * 16
                    agg_v[pl.ds(off, 16)] = jnp.maximum(
                        agg_v[pl.ds(off, 16)], msg)
            return 0
        lax.fori_loop(0, K // 16, group_body, 0)
        return 0

    lax.fori_loop(0, nchunks, chunk_body, 0)

    pltpu.sync_copy(agg_v.at[pl.ds(0, NPW * EMB)],
                    out_hbm.at[pl.ds(base * EMB, NPW * EMB)])


_sc_agg = pl.kernel(
    _sc_agg_body,
    out_type=jax.ShapeDtypeStruct((NPAD * EMB,), jnp.float32),
    mesh=plsc.VectorSubcoreMesh(core_axis_name="c", subcore_axis_name="s"),
    scratch_types=[
        pltpu.VMEM((K,), jnp.int32),
        pltpu.VMEM((K,), jnp.float32),
        pltpu.VMEM((K, EMB), jnp.float32),
        pltpu.VMEM((48,), jnp.int32),
        pltpu.VMEM((K, EMB), jnp.float32),
        pltpu.VMEM((CAP * EMB,), jnp.float32),
        pltpu.SemaphoreType.DMA,
    ],
)


def _mm_body(x_ref, w_ref, o_ref):
    o_ref[...] = jnp.dot(x_ref[...], w_ref[...],
                         preferred_element_type=jnp.float32)


_mm = pl.pallas_call(
    _mm_body, out_shape=jax.ShapeDtypeStruct((N, EMB), jnp.float32))

_EB = 4000

_mm_ea = pl.pallas_call(
    _mm_body,
    grid=(E // _EB,),
    in_specs=[pl.BlockSpec((_EB, 4), lambda i: (i, 0)),
              pl.BlockSpec((4, EMB), lambda i: (0, 0))],
    out_specs=pl.BlockSpec((_EB, EMB), lambda i: (i, 0)),
    out_shape=jax.ShapeDtypeStruct((E, EMB), jnp.float32))


def _tc_step_body(has_next, bi_c_ref, bi_r_ref, x_ref, xg_ref, raw_ref,
                  bm_ref, wax_ref, wag_ref, waa_ref, ba_ref, wgm_ref,
                  bgm_ref, wgf_ref, bgf_ref, wtp_ref, wtg_ref, bt_ref,
                  wmn_ref, *out_refs):
    x = x_ref[...]
    xg = xg_ref[...]
    raw = raw_ref[...][:N]
    bi_c = bi_c_ref[...]
    bi_r = bi_r_ref[...]
    oh = bi_c == lax.broadcasted_iota(jnp.int32, (1, G), 1)
    ohf = oh.astype(jnp.float32)
    oht = (bi_r == lax.broadcasted_iota(jnp.int32, (G, 1), 0))
    ohtf = oht.astype(jnp.float32)

    agg = jnp.where(jnp.isfinite(raw), _leaky(raw + bm_ref[...]), 0.0)
    brd = jnp.dot(ohf, jnp.dot(xg, wag_ref[...],
                               preferred_element_type=jnp.float32),
                  preferred_element_type=jnp.float32)
    u = (jnp.dot(x, wax_ref[...], preferred_element_type=jnp.float32)
         + jnp.dot(agg, waa_ref[...], preferred_element_type=jnp.float32)
         + brd + ba_ref[...])
    x2 = _leaky(u) + x

    gl = jnp.sum(x2 * wgm_ref[...], axis=1, keepdims=True) + bgm_ref[...]
    gmax = jnp.max(jnp.where(oh, gl, NEG), axis=0, keepdims=True)
    gmax_b = jnp.max(jnp.where(oh, gmax, NEG), axis=1, keepdims=True)
    ge = jnp.exp(gl - gmax_b)
    gsum = jnp.sum(jnp.where(oh, ge, 0.0), axis=0, keepdims=True)
    gsum_b = jnp.max(jnp.where(oh, gsum, 0.0), axis=1, keepdims=True)
    alpha = ge / (gsum_b + 1e-16)
    feat = _leaky(jnp.dot(x2, wgf_ref[...],
                          preferred_element_type=jnp.float32) + bgf_ref[...])
    pooled = jnp.dot(ohtf, alpha * feat,
                     preferred_element_type=jnp.float32)
    xg2 = _leaky(jnp.dot(pooled, wtp_ref[...],
                         preferred_element_type=jnp.float32)
                 + jnp.dot(xg, wtg_ref[...],
                           preferred_element_type=jnp.float32)
                 + bt_ref[...]) + xg

    out_refs[0][...] = x2
    out_refs[1][...] = xg2
    if has_next:
        out_refs[2][...] = jnp.dot(x2, wmn_ref[...],
                                   preferred_element_type=jnp.float32)


_tc_step = pl.pallas_call(
    functools.partial(_tc_step_body, True),
    out_shape=[jax.ShapeDtypeStruct((N, EMB), jnp.float32),
               jax.ShapeDtypeStruct((G, EMB), jnp.float32),
               jax.ShapeDtypeStruct((N, EMB), jnp.float32)])


def kernel(x, step_idx, edge_attr, edge_index, batch_ind, num_graphs,
           Wm, bm, Wa, ba, Wgm, bgm, Wgf, bgf, Wt, bt):
    src = edge_index[0].astype(jnp.int32)
    dst = edge_index[1].astype(jnp.int32)
    perm = jnp.argsort(dst)
    src_s = src[perm]
    dst_s = dst[perm]
    dst_f = dst_s.astype(jnp.float32)
    eap = edge_attr[perm]
    bounds = jnp.arange(33, dtype=jnp.int32) * NPW
    starts = jnp.searchsorted(dst_s, bounds).astype(jnp.int32)
    starts = jnp.pad(starts, (0, 15), constant_values=E)
    bi_c = batch_ind.astype(jnp.int32).reshape(N, 1)
    bi_r = batch_ind.astype(jnp.int32).reshape(1, N)

    xg0 = jnp.zeros((G, EMB), jnp.float32)
    xm0 = _mm(x, Wm[0][:EMB])
    wm_next = jnp.roll(Wm, -1, axis=0)[:, :EMB]

    def step(carry, ws):
        xc, xgc, xmc = carry
        wmi, wai, bai, bmi, wgmi, bgmi, wgfi, bgfi, wti, bti, wmn = ws
        eam = _mm_ea(eap, wmi[EMB:])
        raw = _sc_agg(xmc, src_s, dst_f, eam, starts).reshape(NPAD, EMB)
        x2, xg2, xmn = _tc_step(
            bi_c, bi_r, xc, xgc, raw,
            bmi.reshape(1, EMB),
            wai[:EMB], wai[EMB:2 * EMB], wai[2 * EMB:],
            bai.reshape(1, EMB),
            wgmi.reshape(1, EMB),
            bgmi.reshape(1, 1),
            wgfi, bgfi.reshape(1, EMB),
            wti[:EMB], wti[EMB:],
            bti.reshape(1, EMB),
            wmn)
        return (x2, xg2, xmn), None

    (xf, xgf, _), _ = lax.scan(
        step, (x, xg0, xm0),
        (Wm, Wa, ba, bm, Wgm, bgm, Wgf, bgf, Wt, bt, wm_next))
    return (xf, xgf)


# speedup vs baseline: 3.1937x; 1.3610x over previous
import functools

import jax
import jax.numpy as jnp
from jax import lax
from jax.experimental import pallas as pl
from jax.experimental.pallas import tpu as pltpu
from jax.experimental.pallas import tpu_sc as plsc

N = 10000
EMB = 128
E = 320000
G = 16
STEPS = 3

NW = 32
NPW = (N + NW - 1) // NW
NPAD = NW * NPW
CAP = NPW + 1
K = 128
NEG = float("-inf")
NEGBIG = float(-3e38)


def _leaky(z):
    return jnp.where(z >= 0, z, 0.01 * z)


def _sc_agg_body(xm_hbm, src_hbm, dstf_hbm, eam_hbm, starts_hbm,
                 out_hbm,
                 idx_v, dstf_v, eam_v, starts_v, rows_v, agg_v, sem):
    wid = lax.axis_index("s") * 2 + lax.axis_index("c")
    base = wid * NPW

    pltpu.sync_copy(starts_hbm, starts_v)

    iot = lax.broadcasted_iota(jnp.int32, (16,), 0)
    s0 = starts_v[pl.ds(0, 16)]
    s1 = starts_v[pl.ds(16, 16)]
    s2 = starts_v[pl.ds(32, 16)]

    vals = ([s0[j] for j in range(16)] + [s1[j] for j in range(16)]
            + [s2[j] for j in range(16)])

    def _pick(pos):
        r = vals[0]
        for idx in range(1, 33):
            r = jnp.where(pos == idx, vals[idx], r)
        return r

    lo = _pick(wid)
    hi = _pick(wid + 1)

    def init_body(i, _):
        agg_v[pl.ds(i * 16, 16)] = jnp.full((16,), NEG, jnp.float32)
        return 0
    lax.fori_loop(0, CAP * EMB // 16, init_body, 0)

    lo_al = (lo // K) * K
    nchunks = (hi - lo_al + K - 1) // K

    def chunk_body(ci, carry):
        e0 = lo_al + ci * K
        pltpu.sync_copy(src_hbm.at[pl.ds(e0, K)], idx_v)
        pltpu.sync_copy(dstf_hbm.at[pl.ds(e0, K)], dstf_v)
        gat = pltpu.async_copy(xm_hbm.at[idx_v], rows_v, sem)
        pltpu.sync_copy(eam_hbm.at[pl.ds(e0, K)], eam_v)
        gat.wait()

        def group_body(gi, gcarry):
            gb = gi * 16
            evec = e0 + gb + iot
            dvec = dstf_v[pl.ds(gb, 16)].astype(jnp.int32)
            validv = (evec >= lo) & (evec < hi)
            rbv = jnp.where(validv, dvec - base, NPW) * EMB
            cur = gcarry[0]
            accs = gcarry[1:]
            for j in range(16):
                rb = rbv[j]
                k = gb + j
                msgs = tuple(rows_v[k, pl.ds(f * 16, 16)]
                             + eam_v[k, pl.ds(f * 16, 16)]
                             for f in range(EMB // 16))

                @pl.when(rb != cur)
                def _flush(c=cur, a=accs):
                    for f in range(EMB // 16):
                        off = c + f * 16
                        agg_v[pl.ds(off, 16)] = jnp.maximum(
                            agg_v[pl.ds(off, 16)], a[f])

                s = jnp.where(rb == cur, jnp.float32(1.0), jnp.float32(0.0))
                nb1 = NEGBIG * (jnp.float32(1.0) - s)
                accs = tuple(jnp.maximum(m, a * s + nb1)
                             for a, m in zip(accs, msgs))
                cur = rb
            return (cur,) + accs
        return lax.fori_loop(0, K // 16, group_body, carry)

    init = ((jnp.int32(NPW * EMB),)
            + tuple(jnp.full((16,), NEGBIG, jnp.float32)
                    for _ in range(EMB // 16)))
    fin = lax.fori_loop(0, nchunks, chunk_body, init)
    curf = fin[0]
    for f in range(EMB // 16):
        off = curf + f * 16
        agg_v[pl.ds(off, 16)] = jnp.maximum(agg_v[pl.ds(off, 16)],
                                            fin[1 + f])

    pltpu.sync_copy(agg_v.at[pl.ds(0, NPW * EMB)],
                    out_hbm.at[pl.ds(base * EMB, NPW * EMB)])


_sc_agg = pl.kernel(
    _sc_agg_body,
    out_type=jax.ShapeDtypeStruct((NPAD * EMB,), jnp.float32),
    mesh=plsc.VectorSubcoreMesh(core_axis_name="c", subcore_axis_name="s"),
    scratch_types=[
        pltpu.VMEM((K,), jnp.int32),
        pltpu.VMEM((K,), jnp.float32),
        pltpu.VMEM((K, EMB), jnp.float32),
        pltpu.VMEM((48,), jnp.int32),
        pltpu.VMEM((K, EMB), jnp.float32),
        pltpu.VMEM((CAP * EMB,), jnp.float32),
        pltpu.SemaphoreType.DMA,
    ],
)


def _mm_body(x_ref, w_ref, o_ref):
    o_ref[...] = jnp.dot(x_ref[...], w_ref[...],
                         preferred_element_type=jnp.float32)


_mm = pl.pallas_call(
    _mm_body, out_shape=jax.ShapeDtypeStruct((N, EMB), jnp.float32))

_EB = 4000

_mm_ea = pl.pallas_call(
    _mm_body,
    grid=(E // _EB,),
    in_specs=[pl.BlockSpec((_EB, 4), lambda i: (i, 0)),
              pl.BlockSpec((4, EMB), lambda i: (0, 0))],
    out_specs=pl.BlockSpec((_EB, EMB), lambda i: (i, 0)),
    out_shape=jax.ShapeDtypeStruct((E, EMB), jnp.float32))


def _tc_step_body(has_next, bi_c_ref, bi_r_ref, x_ref, xg_ref, raw_ref,
                  bm_ref, wax_ref, wag_ref, waa_ref, ba_ref, wgm_ref,
                  bgm_ref, wgf_ref, bgf_ref, wtp_ref, wtg_ref, bt_ref,
                  wmn_ref, *out_refs):
    x = x_ref[...]
    xg = xg_ref[...]
    raw = raw_ref[...][:N]
    bi_c = bi_c_ref[...]
    bi_r = bi_r_ref[...]
    oh = bi_c == lax.broadcasted_iota(jnp.int32, (1, G), 1)
    ohf = oh.astype(jnp.float32)
    oht = (bi_r == lax.broadcasted_iota(jnp.int32, (G, 1), 0))
    ohtf = oht.astype(jnp.float32)

    agg = jnp.where(jnp.isfinite(raw), _leaky(raw + bm_ref[...]), 0.0)
    brd = jnp.dot(ohf, jnp.dot(xg, wag_ref[...],
                               preferred_element_type=jnp.float32),
                  preferred_element_type=jnp.float32)
    u = (jnp.dot(x, wax_ref[...], preferred_element_type=jnp.float32)
         + jnp.dot(agg, waa_ref[...], preferred_element_type=jnp.float32)
         + brd + ba_ref[...])
    x2 = _leaky(u) + x

    gl = jnp.sum(x2 * wgm_ref[...], axis=1, keepdims=True) + bgm_ref[...]
    gmax = jnp.max(jnp.where(oh, gl, NEG), axis=0, keepdims=True)
    gmax_b = jnp.max(jnp.where(oh, gmax, NEG), axis=1, keepdims=True)
    ge = jnp.exp(gl - gmax_b)
    gsum = jnp.sum(jnp.where(oh, ge, 0.0), axis=0, keepdims=True)
    gsum_b = jnp.max(jnp.where(oh, gsum, 0.0), axis=1, keepdims=True)
    alpha = ge / (gsum_b + 1e-16)
    feat = _leaky(jnp.dot(x2, wgf_ref[...],
                          preferred_element_type=jnp.float32) + bgf_ref[...])
    pooled = jnp.dot(ohtf, alpha * feat,
                     preferred_element_type=jnp.float32)
    xg2 = _leaky(jnp.dot(pooled, wtp_ref[...],
                         preferred_element_type=jnp.float32)
                 + jnp.dot(xg, wtg_ref[...],
                           preferred_element_type=jnp.float32)
                 + bt_ref[...]) + xg

    out_refs[0][...] = x2
    out_refs[1][...] = xg2
    if has_next:
        out_refs[2][...] = jnp.dot(x2, wmn_ref[...],
                                   preferred_element_type=jnp.float32)


_tc_step = pl.pallas_call(
    functools.partial(_tc_step_body, True),
    out_shape=[jax.ShapeDtypeStruct((N, EMB), jnp.float32),
               jax.ShapeDtypeStruct((G, EMB), jnp.float32),
               jax.ShapeDtypeStruct((N, EMB), jnp.float32)])


def kernel(x, step_idx, edge_attr, edge_index, batch_ind, num_graphs,
           Wm, bm, Wa, ba, Wgm, bgm, Wgf, bgf, Wt, bt):
    src = edge_index[0].astype(jnp.int32)
    dst = edge_index[1].astype(jnp.int32)
    perm = jnp.argsort(dst)
    src_s = src[perm]
    dst_s = dst[perm]
    dst_f = dst_s.astype(jnp.float32)
    eap = edge_attr[perm]
    bounds = jnp.arange(33, dtype=jnp.int32) * NPW
    starts = jnp.searchsorted(dst_s, bounds).astype(jnp.int32)
    starts = jnp.pad(starts, (0, 15), constant_values=E)
    bi_c = batch_ind.astype(jnp.int32).reshape(N, 1)
    bi_r = batch_ind.astype(jnp.int32).reshape(1, N)

    xg0 = jnp.zeros((G, EMB), jnp.float32)
    xm0 = _mm(x, Wm[0][:EMB])
    wm_next = jnp.roll(Wm, -1, axis=0)[:, :EMB]

    def step(carry, ws):
        xc, xgc, xmc = carry
        wmi, wai, bai, bmi, wgmi, bgmi, wgfi, bgfi, wti, bti, wmn = ws
        eam = _mm_ea(eap, wmi[EMB:])
        raw = _sc_agg(xmc, src_s, dst_f, eam, starts).reshape(NPAD, EMB)
        x2, xg2, xmn = _tc_step(
            bi_c, bi_r, xc, xgc, raw,
            bmi.reshape(1, EMB),
            wai[:EMB], wai[EMB:2 * EMB], wai[2 * EMB:],
            bai.reshape(1, EMB),
            wgmi.reshape(1, EMB),
            bgmi.reshape(1, 1),
            wgfi, bgfi.reshape(1, EMB),
            wti[:EMB], wti[EMB:],
            bti.reshape(1, EMB),
            wmn)
        return (x2, xg2, xmn), None

    (xf, xgf, _), _ = lax.scan(
        step, (x, xg0, xm0),
        (Wm, Wa, ba, bm, Wgm, bgm, Wgf, bgf, Wt, bt, wm_next))
    return (xf, xgf)


# speedup vs baseline: 3.3815x; 1.0588x over previous
import functools

import jax
import jax.numpy as jnp
from jax import lax
from jax.experimental import pallas as pl
from jax.experimental.pallas import tpu as pltpu
from jax.experimental.pallas import tpu_sc as plsc

N = 10000
EMB = 128
E = 320000
G = 16
STEPS = 3

NW = 32
NPW = (N + NW - 1) // NW
NPAD = NW * NPW
CAP = NPW + 1
K = 256
NEG = float("-inf")
NEGBIG = float(-3e38)


def _leaky(z):
    return jnp.where(z >= 0, z, 0.01 * z)


def _sc_agg_body(xm_hbm, src_hbm, dstf_hbm, eam_hbm, starts_hbm,
                 out_hbm,
                 idx_v, dstf_v, eam_v, starts_v, rows_v, agg_v, sem):
    wid = lax.axis_index("s") * 2 + lax.axis_index("c")
    base = wid * NPW

    pltpu.sync_copy(starts_hbm, starts_v)

    iot = lax.broadcasted_iota(jnp.int32, (16,), 0)
    s0 = starts_v[pl.ds(0, 16)]
    s1 = starts_v[pl.ds(16, 16)]
    s2 = starts_v[pl.ds(32, 16)]

    vals = ([s0[j] for j in range(16)] + [s1[j] for j in range(16)]
            + [s2[j] for j in range(16)])

    def _pick(pos):
        r = vals[0]
        for idx in range(1, 33):
            r = jnp.where(pos == idx, vals[idx], r)
        return r

    lo = _pick(wid)
    hi = _pick(wid + 1)

    def init_body(i, _):
        agg_v[pl.ds(i * 16, 16)] = jnp.full((16,), NEG, jnp.float32)
        return 0
    lax.fori_loop(0, CAP * EMB // 16, init_body, 0)

    lo_al = (lo // K) * K
    nchunks = (hi - lo_al + K - 1) // K

    def chunk_body(ci, carry):
        e0 = lo_al + ci * K
        pltpu.sync_copy(src_hbm.at[pl.ds(e0, K)], idx_v)
        pltpu.sync_copy(dstf_hbm.at[pl.ds(e0, K)], dstf_v)
        gats = [pltpu.async_copy(xm_hbm.at[idx_v.at[pl.ds(h * 128, 128)]],
                                 rows_v.at[pl.ds(h * 128, 128)], sem)
                for h in range(K // 128)]
        pltpu.sync_copy(eam_hbm.at[pl.ds(e0, K)], eam_v)
        for g in gats:
            g.wait()

        def group_body(gi, gcarry):
            gb = gi * 16
            evec = e0 + gb + iot
            dvec = dstf_v[pl.ds(gb, 16)].astype(jnp.int32)
            validv = (evec >= lo) & (evec < hi)
            rbv = jnp.where(validv, dvec - base, NPW) * EMB
            cur = gcarry[0]
            accs = gcarry[1:]
            for j in range(16):
                rb = rbv[j]
                k = gb + j
                msgs = tuple(rows_v[k, pl.ds(f * 16, 16)]
                             + eam_v[k, pl.ds(f * 16, 16)]
                             for f in range(EMB // 16))

                @pl.when(rb != cur)
                def _flush(c=cur, a=accs):
                    for f in range(EMB // 16):
                        off = c + f * 16
                        agg_v[pl.ds(off, 16)] = jnp.maximum(
                            agg_v[pl.ds(off, 16)], a[f])

                s = jnp.where(rb == cur, jnp.float32(1.0), jnp.float32(0.0))
                nb1 = NEGBIG * (jnp.float32(1.0) - s)
                accs = tuple(jnp.maximum(m, a * s + nb1)
                             for a, m in zip(accs, msgs))
                cur = rb
            return (cur,) + accs
        return lax.fori_loop(0, K // 16, group_body, carry)

    init = ((jnp.int32(NPW * EMB),)
            + tuple(jnp.full((16,), NEGBIG, jnp.float32)
                    for _ in range(EMB // 16)))
    fin = lax.fori_loop(0, nchunks, chunk_body, init)
    curf = fin[0]
    for f in range(EMB // 16):
        off = curf + f * 16
        agg_v[pl.ds(off, 16)] = jnp.maximum(agg_v[pl.ds(off, 16)],
                                            fin[1 + f])

    pltpu.sync_copy(agg_v.at[pl.ds(0, NPW * EMB)],
                    out_hbm.at[pl.ds(base * EMB, NPW * EMB)])


_sc_agg = pl.kernel(
    _sc_agg_body,
    out_type=jax.ShapeDtypeStruct((NPAD * EMB,), jnp.float32),
    mesh=plsc.VectorSubcoreMesh(core_axis_name="c", subcore_axis_name="s"),
    scratch_types=[
        pltpu.VMEM((K,), jnp.int32),
        pltpu.VMEM((K,), jnp.float32),
        pltpu.VMEM((K, EMB), jnp.float32),
        pltpu.VMEM((48,), jnp.int32),
        pltpu.VMEM((K, EMB), jnp.float32),
        pltpu.VMEM((CAP * EMB,), jnp.float32),
        pltpu.SemaphoreType.DMA,
    ],
)


def _mm_body(x_ref, w_ref, o_ref):
    o_ref[...] = jnp.dot(x_ref[...], w_ref[...],
                         preferred_element_type=jnp.float32)


_mm = pl.pallas_call(
    _mm_body, out_shape=jax.ShapeDtypeStruct((N, EMB), jnp.float32))

_EB = 4000

_mm_ea = pl.pallas_call(
    _mm_body,
    grid=(E // _EB,),
    in_specs=[pl.BlockSpec((_EB, 4), lambda i: (i, 0)),
              pl.BlockSpec((4, EMB), lambda i: (0, 0))],
    out_specs=pl.BlockSpec((_EB, EMB), lambda i: (i, 0)),
    out_shape=jax.ShapeDtypeStruct((E, EMB), jnp.float32))


def _tc_step_body(has_next, bi_c_ref, bi_r_ref, x_ref, xg_ref, raw_ref,
                  bm_ref, wax_ref, wag_ref, waa_ref, ba_ref, wgm_ref,
                  bgm_ref, wgf_ref, bgf_ref, wtp_ref, wtg_ref, bt_ref,
                  wmn_ref, *out_refs):
    x = x_ref[...]
    xg = xg_ref[...]
    raw = raw_ref[...][:N]
    bi_c = bi_c_ref[...]
    bi_r = bi_r_ref[...]
    oh = bi_c == lax.broadcasted_iota(jnp.int32, (1, G), 1)
    ohf = oh.astype(jnp.float32)
    oht = (bi_r == lax.broadcasted_iota(jnp.int32, (G, 1), 0))
    ohtf = oht.astype(jnp.float32)

    agg = jnp.where(jnp.isfinite(raw), _leaky(raw + bm_ref[...]), 0.0)
    brd = jnp.dot(ohf, jnp.dot(xg, wag_ref[...],
                               preferred_element_type=jnp.float32),
                  preferred_element_type=jnp.float32)
    u = (jnp.dot(x, wax_ref[...], preferred_element_type=jnp.float32)
         + jnp.dot(agg, waa_ref[...], preferred_element_type=jnp.float32)
         + brd + ba_ref[...])
    x2 = _leaky(u) + x

    gl = jnp.sum(x2 * wgm_ref[...], axis=1, keepdims=True) + bgm_ref[...]
    gmax = jnp.max(jnp.where(oh, gl, NEG), axis=0, keepdims=True)
    gmax_b = jnp.max(jnp.where(oh, gmax, NEG), axis=1, keepdims=True)
    ge = jnp.exp(gl - gmax_b)
    gsum = jnp.sum(jnp.where(oh, ge, 0.0), axis=0, keepdims=True)
    gsum_b = jnp.max(jnp.where(oh, gsum, 0.0), axis=1, keepdims=True)
    alpha = ge / (gsum_b + 1e-16)
    feat = _leaky(jnp.dot(x2, wgf_ref[...],
                          preferred_element_type=jnp.float32) + bgf_ref[...])
    pooled = jnp.dot(ohtf, alpha * feat,
                     preferred_element_type=jnp.float32)
    xg2 = _leaky(jnp.dot(pooled, wtp_ref[...],
                         preferred_element_type=jnp.float32)
                 + jnp.dot(xg, wtg_ref[...],
                           preferred_element_type=jnp.float32)
                 + bt_ref[...]) + xg

    out_refs[0][...] = x2
    out_refs[1][...] = xg2
    if has_next:
        out_refs[2][...] = jnp.dot(x2, wmn_ref[...],
                                   preferred_element_type=jnp.float32)


_tc_step = pl.pallas_call(
    functools.partial(_tc_step_body, True),
    out_shape=[jax.ShapeDtypeStruct((N, EMB), jnp.float32),
               jax.ShapeDtypeStruct((G, EMB), jnp.float32),
               jax.ShapeDtypeStruct((N, EMB), jnp.float32)])


def kernel(x, step_idx, edge_attr, edge_index, batch_ind, num_graphs,
           Wm, bm, Wa, ba, Wgm, bgm, Wgf, bgf, Wt, bt):
    src = edge_index[0].astype(jnp.int32)
    dst = edge_index[1].astype(jnp.int32)
    perm = jnp.argsort(dst)
    src_s = src[perm]
    dst_s = dst[perm]
    dst_f = dst_s.astype(jnp.float32)
    eap = edge_attr[perm]
    bounds = jnp.arange(33, dtype=jnp.int32) * NPW
    starts = jnp.searchsorted(dst_s, bounds).astype(jnp.int32)
    starts = jnp.pad(starts, (0, 15), constant_values=E)
    bi_c = batch_ind.astype(jnp.int32).reshape(N, 1)
    bi_r = batch_ind.astype(jnp.int32).reshape(1, N)

    xg0 = jnp.zeros((G, EMB), jnp.float32)
    xm0 = _mm(x, Wm[0][:EMB])
    wm_next = jnp.roll(Wm, -1, axis=0)[:, :EMB]

    def step(carry, ws):
        xc, xgc, xmc = carry
        wmi, wai, bai, bmi, wgmi, bgmi, wgfi, bgfi, wti, bti, wmn = ws
        eam = _mm_ea(eap, wmi[EMB:])
        raw = _sc_agg(xmc, src_s, dst_f, eam, starts).reshape(NPAD, EMB)
        x2, xg2, xmn = _tc_step(
            bi_c, bi_r, xc, xgc, raw,
            bmi.reshape(1, EMB),
            wai[:EMB], wai[EMB:2 * EMB], wai[2 * EMB:],
            bai.reshape(1, EMB),
            wgmi.reshape(1, EMB),
            bgmi.reshape(1, 1),
            wgfi, bgfi.reshape(1, EMB),
            wti[:EMB], wti[EMB:],
            bti.reshape(1, EMB),
            wmn)
        return (x2, xg2, xmn), None

    (xf, xgf, _), _ = lax.scan(
        step, (x, xg0, xm0),
        (Wm, Wa, ba, bm, Wgm, bgm, Wgf, bgf, Wt, bt, wm_next))
    return (xf, xgf)


# speedup vs baseline: 3.3859x; 1.0013x over previous
import functools

import jax
import jax.numpy as jnp
from jax import lax
from jax.experimental import pallas as pl
from jax.experimental.pallas import tpu as pltpu
from jax.experimental.pallas import tpu_sc as plsc

N = 10000
EMB = 128
E = 320000
G = 16
STEPS = 3

NW = 32
NPW = (N + NW - 1) // NW
NPAD = NW * NPW
CAP = NPW + 1
K = 256
NEG = float("-inf")
NEGBIG = float(-3e38)


def _leaky(z):
    return jnp.where(z >= 0, z, 0.01 * z)


def _sc_agg_body(xm_hbm, src_hbm, dstf_hbm, eam_hbm, starts_hbm,
                 out_hbm,
                 idx_v, dstf_v, eam_v, starts_v, rows_v, agg_v, sem):
    wid = lax.axis_index("s") * 2 + lax.axis_index("c")
    base = wid * NPW

    pltpu.sync_copy(starts_hbm, starts_v)

    iot = lax.broadcasted_iota(jnp.int32, (16,), 0)
    s0 = starts_v[pl.ds(0, 16)]
    s1 = starts_v[pl.ds(16, 16)]
    s2 = starts_v[pl.ds(32, 16)]

    vals = ([s0[j] for j in range(16)] + [s1[j] for j in range(16)]
            + [s2[j] for j in range(16)])

    def _pick(pos):
        r = vals[0]
        for idx in range(1, 33):
            r = jnp.where(pos == idx, vals[idx], r)
        return r

    lo = _pick(wid)
    hi = _pick(wid + 1)

    def init_body(i, _):
        agg_v[pl.ds(i * 16, 16)] = jnp.full((16,), NEG, jnp.float32)
        return 0
    lax.fori_loop(0, CAP * EMB // 16, init_body, 0)

    lo_al = (lo // K) * K
    nchunks = (hi - lo_al + K - 1) // K

    def chunk_body(ci, carry):
        e0 = lo_al + ci * K
        pltpu.sync_copy(src_hbm.at[pl.ds(e0, K)], idx_v)
        pltpu.sync_copy(dstf_hbm.at[pl.ds(e0, K)], dstf_v)
        gats = [pltpu.async_copy(xm_hbm.at[idx_v.at[pl.ds(h * 128, 128)]],
                                 rows_v.at[pl.ds(h * 128, 128)], sem)
                for h in range(K // 128)]
        pltpu.sync_copy(eam_hbm.at[pl.ds(e0, K)], eam_v)
        for g in gats:
            g.wait()

        def group_body(gi, gcarry):
            gb = gi * 16
            evec = e0 + gb + iot
            dvec = dstf_v[pl.ds(gb, 16)].astype(jnp.int32)
            validv = (evec >= lo) & (evec < hi)
            rbv = jnp.where(validv, dvec - base, NPW) * EMB
            cur = gcarry[0]
            accs = gcarry[1:]
            for j in range(16):
                rb = rbv[j]
                k = gb + j
                msgs = tuple(rows_v[k, pl.ds(f * 16, 16)]
                             + eam_v[k, pl.ds(f * 16, 16)]
                             for f in range(EMB // 16))

                @pl.when(rb != cur)
                def _flush(c=cur, a=accs):
                    for f in range(EMB // 16):
                        off = c + f * 16
                        agg_v[pl.ds(off, 16)] = jnp.maximum(
                            agg_v[pl.ds(off, 16)], a[f])

                s = jnp.where(rb == cur, jnp.float32(1.0), jnp.float32(0.0))
                nb1 = NEGBIG * (jnp.float32(1.0) - s)
                accs = tuple(jnp.maximum(m, a * s + nb1)
                             for a, m in zip(accs, msgs))
                cur = rb
            return (cur,) + accs
        return lax.fori_loop(0, K // 16, group_body, carry)

    init = ((jnp.int32(NPW * EMB),)
            + tuple(jnp.full((16,), NEGBIG, jnp.float32)
                    for _ in range(EMB // 16)))
    fin = lax.fori_loop(0, nchunks, chunk_body, init)
    curf = fin[0]
    for f in range(EMB // 16):
        off = curf + f * 16
        agg_v[pl.ds(off, 16)] = jnp.maximum(agg_v[pl.ds(off, 16)],
                                            fin[1 + f])

    pltpu.sync_copy(agg_v.at[pl.ds(0, NPW * EMB)],
                    out_hbm.at[pl.ds(base * EMB, NPW * EMB)])


_sc_agg = pl.kernel(
    _sc_agg_body,
    out_type=jax.ShapeDtypeStruct((NPAD * EMB,), jnp.float32),
    mesh=plsc.VectorSubcoreMesh(core_axis_name="c", subcore_axis_name="s"),
    scratch_types=[
        pltpu.VMEM((K,), jnp.int32),
        pltpu.VMEM((K,), jnp.float32),
        pltpu.VMEM((K, EMB), jnp.float32),
        pltpu.VMEM((48,), jnp.int32),
        pltpu.VMEM((K, EMB), jnp.float32),
        pltpu.VMEM((CAP * EMB,), jnp.float32),
        pltpu.SemaphoreType.DMA,
    ],
)


def _mm_body(x_ref, w_ref, o_ref):
    o_ref[...] = jnp.dot(x_ref[...], w_ref[...],
                         preferred_element_type=jnp.float32)


_mm = pl.pallas_call(
    _mm_body, out_shape=jax.ShapeDtypeStruct((N, EMB), jnp.float32))

_EB = 4000

_mm_ea = pl.pallas_call(
    _mm_body,
    grid=(E // _EB,),
    in_specs=[pl.BlockSpec((_EB, 4), lambda i: (i, 0)),
              pl.BlockSpec((4, EMB), lambda i: (0, 0))],
    out_specs=pl.BlockSpec((_EB, EMB), lambda i: (i, 0)),
    out_shape=jax.ShapeDtypeStruct((E, EMB), jnp.float32))


_R = 2000
_NB = N // _R


def _oh_of(bi):
    return bi == lax.broadcasted_iota(jnp.int32, (1, G), 1)


def _tc_a_body(bi_ref, x_ref, raw_ref, xg_ref, bm_ref, wax_ref, wag_ref,
               waa_ref, ba_ref, wgm_ref, bgm_ref, wgf_ref, bgf_ref, wmn_ref,
               x2_ref, xmn_ref, feat_ref, gl_ref, gmax_ref):
    i = pl.program_id(0)
    x = x_ref[...]
    raw = raw_ref[...]
    oh = _oh_of(bi_ref[...])
    ohf = oh.astype(jnp.float32)
    agg = jnp.where(jnp.isfinite(raw), _leaky(raw + bm_ref[...]), 0.0)
    gxg = jnp.dot(xg_ref[...], wag_ref[...],
                  preferred_element_type=jnp.float32)
    u = (jnp.dot(x, wax_ref[...], preferred_element_type=jnp.float32)
         + jnp.dot(agg, waa_ref[...], preferred_element_type=jnp.float32)
         + jnp.dot(ohf, gxg, preferred_element_type=jnp.float32)
         + ba_ref[...])
    x2 = _leaky(u) + x
    gl = jnp.sum(x2 * wgm_ref[...], axis=1, keepdims=True) + bgm_ref[...]
    x2_ref[...] = x2
    xmn_ref[...] = jnp.dot(x2, wmn_ref[...],
                           preferred_element_type=jnp.float32)
    feat_ref[...] = _leaky(jnp.dot(x2, wgf_ref[...],
                                   preferred_element_type=jnp.float32)
                           + bgf_ref[...])
    gl_ref[...] = gl

    @pl.when(i == 0)
    def _init():
        gmax_ref[...] = jnp.full((1, G), NEG, jnp.float32)
    gmax_ref[...] = jnp.maximum(
        gmax_ref[...],
        jnp.max(jnp.where(oh, gl, NEG), axis=0, keepdims=True))


def _full(shape):
    return pl.BlockSpec(shape, lambda i: tuple(0 for _ in shape))


def _rows(shape):
    return pl.BlockSpec(shape, lambda i: (i, 0))


_tc_a = pl.pallas_call(
    _tc_a_body,
    grid=(_NB,),
    in_specs=[_rows((_R, 1)), _rows((_R, EMB)), _rows((_R, EMB)),
              _full((G, EMB)), _full((1, EMB)), _full((EMB, EMB)),
              _full((EMB, EMB)), _full((EMB, EMB)), _full((1, EMB)),
              _full((1, EMB)), _full((1, 1)), _full((EMB, EMB)),
              _full((1, EMB)), _full((EMB, EMB))],
    out_specs=[_rows((_R, EMB)), _rows((_R, EMB)), _rows((_R, EMB)),
               _rows((_R, 1)), pl.BlockSpec((1, G), lambda i: (0, 0))],
    out_shape=[jax.ShapeDtypeStruct((N, EMB), jnp.float32),
               jax.ShapeDtypeStruct((N, EMB), jnp.float32),
               jax.ShapeDtypeStruct((N, EMB), jnp.float32),
               jax.ShapeDtypeStruct((N, 1), jnp.float32),
               jax.ShapeDtypeStruct((1, G), jnp.float32)])


def _tc_b_body(bi_ref, gl_ref, gmax_ref, gsum_ref):
    i = pl.program_id(0)
    oh = _oh_of(bi_ref[...])
    gmax_b = jnp.max(jnp.where(oh, gmax_ref[...], NEG), axis=1,
                     keepdims=True)
    ge = jnp.exp(gl_ref[...] - gmax_b)

    @pl.when(i == 0)
    def _init():
        gsum_ref[...] = jnp.zeros((1, G), jnp.float32)
    gsum_ref[...] += jnp.sum(jnp.where(oh, ge, 0.0), axis=0, keepdims=True)


_tc_b = pl.pallas_call(
    _tc_b_body,
    grid=(_NB,),
    in_specs=[_rows((_R, 1)), _rows((_R, 1)), _full((1, G))],
    out_specs=pl.BlockSpec((1, G), lambda i: (0, 0)),
    out_shape=jax.ShapeDtypeStruct((1, G), jnp.float32))


def _tc_c_body(bi_c_ref, gl_ref, feat_ref, gmax_ref, gsum_ref,
               pooled_ref):
    i = pl.program_id(0)
    oh = _oh_of(bi_c_ref[...])
    ohf = oh.astype(jnp.float32)
    gmax_b = jnp.max(jnp.where(oh, gmax_ref[...], NEG), axis=1,
                     keepdims=True)
    ge = jnp.exp(gl_ref[...] - gmax_b)
    gsum_b = jnp.max(jnp.where(oh, gsum_ref[...], 0.0), axis=1,
                     keepdims=True)
    alpha = ge / (gsum_b + 1e-16)

    @pl.when(i == 0)
    def _init():
        pooled_ref[...] = jnp.zeros((G, EMB), jnp.float32)
    pooled_ref[...] += lax.dot_general(
        ohf, alpha * feat_ref[...],
        dimension_numbers=(((0,), (0,)), ((), ())),
        preferred_element_type=jnp.float32)


_tc_c = pl.pallas_call(
    _tc_c_body,
    grid=(_NB,),
    in_specs=[_rows((_R, 1)), _rows((_R, 1)), _rows((_R, EMB)),
              _full((1, G)), _full((1, G))],
    out_specs=pl.BlockSpec((G, EMB), lambda i: (0, 0)),
    out_shape=jax.ShapeDtypeStruct((G, EMB), jnp.float32))


def _tc_d_body(pooled_ref, xg_ref, wtp_ref, wtg_ref, bt_ref, xgo_ref):
    xg = xg_ref[...]
    xgo_ref[...] = _leaky(
        jnp.dot(pooled_ref[...], wtp_ref[...],
                preferred_element_type=jnp.float32)
        + jnp.dot(xg, wtg_ref[...], preferred_element_type=jnp.float32)
        + bt_ref[...]) + xg


_tc_d = pl.pallas_call(
    _tc_d_body, out_shape=jax.ShapeDtypeStruct((G, EMB), jnp.float32))


def kernel(x, step_idx, edge_attr, edge_index, batch_ind, num_graphs,
           Wm, bm, Wa, ba, Wgm, bgm, Wgf, bgf, Wt, bt):
    src = edge_index[0].astype(jnp.int32)
    dst = edge_index[1].astype(jnp.int32)
    perm = jnp.argsort(dst)
    src_s = src[perm]
    dst_s = dst[perm]
    dst_f = dst_s.astype(jnp.float32)
    eap = edge_attr[perm]
    bounds = jnp.arange(33, dtype=jnp.int32) * NPW
    starts = jnp.searchsorted(dst_s, bounds).astype(jnp.int32)
    starts = jnp.pad(starts, (0, 15), constant_values=E)
    bi_c = batch_ind.astype(jnp.int32).reshape(N, 1)

    xg0 = jnp.zeros((G, EMB), jnp.float32)
    xm0 = _mm(x, Wm[0][:EMB])
    wm_next = jnp.roll(Wm, -1, axis=0)[:, :EMB]

    def step(carry, ws):
        xc, xgc, xmc = carry
        wmi, wai, bai, bmi, wgmi, bgmi, wgfi, bgfi, wti, bti, wmn = ws
        eam = _mm_ea(eap, wmi[EMB:])
        raw = _sc_agg(xmc, src_s, dst_f, eam, starts).reshape(NPAD, EMB)
        x2, xmn, feat, gl, gmax = _tc_a(
            bi_c, xc, raw, xgc,
            bmi.reshape(1, EMB),
            wai[:EMB], wai[EMB:2 * EMB], wai[2 * EMB:],
            bai.reshape(1, EMB),
            wgmi.reshape(1, EMB),
            bgmi.reshape(1, 1),
            wgfi, bgfi.reshape(1, EMB),
            wmn)
        gsum = _tc_b(bi_c, gl, gmax)
        pooled = _tc_c(bi_c, gl, feat, gmax, gsum)
        xg2 = _tc_d(pooled, xgc, wti[:EMB], wti[EMB:], bti.reshape(1, EMB))
        return (x2, xg2, xmn), None

    (xf, xgf, _), _ = lax.scan(
        step, (x, xg0, xm0),
        (Wm, Wa, ba, bm, Wgm, bgm, Wgf, bgf, Wt, bt, wm_next))
    return (xf, xgf)


# speedup vs baseline: 3.3873x; 1.0004x over previous
import jax
import jax.numpy as jnp
from jax import lax
from jax.experimental import pallas as pl
from jax.experimental.pallas import tpu as pltpu
from jax.experimental.pallas import tpu_sc as plsc

N = 10000
EMB = 128
E = 320000
G = 16
STEPS = 3

NW = 32
NPW = (N + NW - 1) // NW
NPAD = NW * NPW
CAP = NPW + 1
K = 256
NEG = float("-inf")
NEGBIG = float(-3e38)


def _leaky(z):
    return jnp.where(z >= 0, z, 0.01 * z)


def _sc_agg_body(xm_hbm, src_hbm, dstf_hbm, eam_hbm, starts_hbm,
                 out_hbm,
                 idx_v, dstf_v, eam_v, starts_v, rows_v, agg_v, sem):
    wid = lax.axis_index("s") * 2 + lax.axis_index("c")
    base = wid * NPW

    pltpu.sync_copy(starts_hbm, starts_v)

    iot = lax.broadcasted_iota(jnp.int32, (16,), 0)
    s0 = starts_v[pl.ds(0, 16)]
    s1 = starts_v[pl.ds(16, 16)]
    s2 = starts_v[pl.ds(32, 16)]

    vals = ([s0[j] for j in range(16)] + [s1[j] for j in range(16)]
            + [s2[j] for j in range(16)])

    def _pick(pos):
        r = vals[0]
        for idx in range(1, 33):
            r = jnp.where(pos == idx, vals[idx], r)
        return r

    lo = _pick(wid)
    hi = _pick(wid + 1)

    def init_body(i, _):
        agg_v[pl.ds(i * 16, 16)] = jnp.full((16,), NEG, jnp.float32)
        return 0
    lax.fori_loop(0, CAP * EMB // 16, init_body, 0)

    lo_al = (lo // K) * K
    nchunks = (hi - lo_al + K - 1) // K

    def chunk_body(ci, carry):
        e0 = lo_al + ci * K
        pltpu.sync_copy(src_hbm.at[pl.ds(e0, K)], idx_v)
        pltpu.sync_copy(dstf_hbm.at[pl.ds(e0, K)], dstf_v)
        gats = [pltpu.async_copy(xm_hbm.at[idx_v.at[pl.ds(h * 128, 128)]],
                                 rows_v.at[pl.ds(h * 128, 128)], sem)
                for h in range(K // 128)]
        pltpu.sync_copy(eam_hbm.at[pl.ds(e0, K)], eam_v)
        for g in gats:
            g.wait()

        def group_body(gi, gcarry):
            gb = gi * 16
            evec = e0 + gb + iot
            dvec = dstf_v[pl.ds(gb, 16)].astype(jnp.int32)
            validv = (evec >= lo) & (evec < hi)
            rbv = jnp.where(validv, dvec - base, NPW) * EMB
            cur = gcarry[0]
            accs = gcarry[1:]
            for j in range(16):
                rb = rbv[j]
                k = gb + j
                msgs = tuple(rows_v[k, pl.ds(f * 16, 16)]
                             + eam_v[k, pl.ds(f * 16, 16)]
                             for f in range(EMB // 16))

                @pl.when(rb != cur)
                def _flush(c=cur, a=accs):
                    for f in range(EMB // 16):
                        off = c + f * 16
                        agg_v[pl.ds(off, 16)] = jnp.maximum(
                            agg_v[pl.ds(off, 16)], a[f])

                s = jnp.where(rb == cur, jnp.float32(1.0), jnp.float32(0.0))
                nb1 = NEGBIG * (jnp.float32(1.0) - s)
                accs = tuple(jnp.maximum(m, a * s + nb1)
                             for a, m in zip(accs, msgs))
                cur = rb
            return (cur,) + accs
        return lax.fori_loop(0, K // 16, group_body, carry)

    init = ((jnp.int32(NPW * EMB),)
            + tuple(jnp.full((16,), NEGBIG, jnp.float32)
                    for _ in range(EMB // 16)))
    fin = lax.fori_loop(0, nchunks, chunk_body, init)
    curf = fin[0]
    for f in range(EMB // 16):
        off = curf + f * 16
        agg_v[pl.ds(off, 16)] = jnp.maximum(agg_v[pl.ds(off, 16)],
                                            fin[1 + f])

    pltpu.sync_copy(agg_v.at[pl.ds(0, NPW * EMB)],
                    out_hbm.at[pl.ds(base * EMB, NPW * EMB)])


_sc_agg = pl.kernel(
    _sc_agg_body,
    out_type=jax.ShapeDtypeStruct((NPAD * EMB,), jnp.float32),
    mesh=plsc.VectorSubcoreMesh(core_axis_name="c", subcore_axis_name="s"),
    scratch_types=[
        pltpu.VMEM((K,), jnp.int32),
        pltpu.VMEM((K,), jnp.float32),
        pltpu.VMEM((K, EMB), jnp.float32),
        pltpu.VMEM((48,), jnp.int32),
        pltpu.VMEM((K, EMB), jnp.float32),
        pltpu.VMEM((CAP * EMB,), jnp.float32),
        pltpu.SemaphoreType.DMA,
    ],
)


def _mm_body(x_ref, w_ref, o_ref):
    o_ref[...] = jnp.dot(x_ref[...], w_ref[...],
                         preferred_element_type=jnp.float32)


_mm = pl.pallas_call(
    _mm_body, out_shape=jax.ShapeDtypeStruct((N, EMB), jnp.float32))

_EB = 4000

_mm_ea = pl.pallas_call(
    _mm_body,
    grid=(E // _EB,),
    in_specs=[pl.BlockSpec((_EB, 4), lambda i: (i, 0)),
              pl.BlockSpec((4, EMB), lambda i: (0, 0))],
    out_specs=pl.BlockSpec((_EB, EMB), lambda i: (i, 0)),
    out_shape=jax.ShapeDtypeStruct((E, EMB), jnp.float32))


_R = 2000
_NB = N // _R


def _oh_of(bi):
    return bi == lax.broadcasted_iota(jnp.int32, (1, G), 1)


def _tc_a_body(bi_ref, x_ref, raw_ref, xg_ref, bm_ref, wax_ref, wag_ref,
               waa_ref, ba_ref, wgm_ref, bgm_ref, wgf_ref, bgf_ref, wmn_ref,
               x2_ref, xmn_ref, feat_ref, gl_ref, gmax_ref):
    i = pl.program_id(0)
    x = x_ref[...]
    raw = raw_ref[...]
    oh = _oh_of(bi_ref[...])
    ohf = oh.astype(jnp.float32)
    agg = jnp.where(jnp.isfinite(raw), _leaky(raw + bm_ref[...]), 0.0)
    gxg = jnp.dot(xg_ref[...], wag_ref[...],
                  preferred_element_type=jnp.float32)
    u = (jnp.dot(x, wax_ref[...], preferred_element_type=jnp.float32)
         + jnp.dot(agg, waa_ref[...], preferred_element_type=jnp.float32)
         + jnp.dot(ohf, gxg, preferred_element_type=jnp.float32)
         + ba_ref[...])
    x2 = _leaky(u) + x
    gl = jnp.sum(x2 * wgm_ref[...], axis=1, keepdims=True) + bgm_ref[...]
    x2_ref[...] = x2
    xmn_ref[...] = jnp.dot(x2, wmn_ref[...],
                           preferred_element_type=jnp.float32)
    feat_ref[...] = _leaky(jnp.dot(x2, wgf_ref[...],
                                   preferred_element_type=jnp.float32)
                           + bgf_ref[...])
    gl_ref[...] = gl

    @pl.when(i == 0)
    def _init():
        gmax_ref[...] = jnp.full((1, G), NEG, jnp.float32)
    gmax_ref[...] = jnp.maximum(
        gmax_ref[...],
        jnp.max(jnp.where(oh, gl, NEG), axis=0, keepdims=True))


def _full(shape):
    return pl.BlockSpec(shape, lambda i: tuple(0 for _ in shape))


def _rows(shape):
    return pl.BlockSpec(shape, lambda i: (i, 0))


_tc_a = pl.pallas_call(
    _tc_a_body,
    grid=(_NB,),
    in_specs=[_rows((_R, 1)), _rows((_R, EMB)), _rows((_R, EMB)),
              _full((G, EMB)), _full((1, EMB)), _full((EMB, EMB)),
              _full((EMB, EMB)), _full((EMB, EMB)), _full((1, EMB)),
              _full((1, EMB)), _full((1, 1)), _full((EMB, EMB)),
              _full((1, EMB)), _full((EMB, EMB))],
    out_specs=[_rows((_R, EMB)), _rows((_R, EMB)), _rows((_R, EMB)),
               _rows((_R, 1)), pl.BlockSpec((1, G), lambda i: (0, 0))],
    out_shape=[jax.ShapeDtypeStruct((N, EMB), jnp.float32),
               jax.ShapeDtypeStruct((N, EMB), jnp.float32),
               jax.ShapeDtypeStruct((N, EMB), jnp.float32),
               jax.ShapeDtypeStruct((N, 1), jnp.float32),
               jax.ShapeDtypeStruct((1, G), jnp.float32)])


def _tc_b_body(bi_ref, gl_ref, gmax_ref, gsum_ref):
    i = pl.program_id(0)
    oh = _oh_of(bi_ref[...])
    gmax_b = jnp.max(jnp.where(oh, gmax_ref[...], NEG), axis=1,
                     keepdims=True)
    ge = jnp.exp(gl_ref[...] - gmax_b)

    @pl.when(i == 0)
    def _init():
        gsum_ref[...] = jnp.zeros((1, G), jnp.float32)
    gsum_ref[...] += jnp.sum(jnp.where(oh, ge, 0.0), axis=0, keepdims=True)


_tc_b = pl.pallas_call(
    _tc_b_body,
    grid=(_NB,),
    in_specs=[_rows((_R, 1)), _rows((_R, 1)), _full((1, G))],
    out_specs=pl.BlockSpec((1, G), lambda i: (0, 0)),
    out_shape=jax.ShapeDtypeStruct((1, G), jnp.float32))


def _tc_c_body(bi_c_ref, gl_ref, feat_ref, gmax_ref, gsum_ref,
               pooled_ref):
    i = pl.program_id(0)
    oh = _oh_of(bi_c_ref[...])
    ohf = oh.astype(jnp.float32)
    gmax_b = jnp.max(jnp.where(oh, gmax_ref[...], NEG), axis=1,
                     keepdims=True)
    ge = jnp.exp(gl_ref[...] - gmax_b)
    gsum_b = jnp.max(jnp.where(oh, gsum_ref[...], 0.0), axis=1,
                     keepdims=True)
    alpha = ge / (gsum_b + 1e-16)

    @pl.when(i == 0)
    def _init():
        pooled_ref[...] = jnp.zeros((G, EMB), jnp.float32)
    pooled_ref[...] += lax.dot_general(
        ohf, alpha * feat_ref[...],
        dimension_numbers=(((0,), (0,)), ((), ())),
        preferred_element_type=jnp.float32)


_tc_c = pl.pallas_call(
    _tc_c_body,
    grid=(_NB,),
    in_specs=[_rows((_R, 1)), _rows((_R, 1)), _rows((_R, EMB)),
              _full((1, G)), _full((1, G))],
    out_specs=pl.BlockSpec((G, EMB), lambda i: (0, 0)),
    out_shape=jax.ShapeDtypeStruct((G, EMB), jnp.float32))


def _tc_d_body(pooled_ref, xg_ref, wtp_ref, wtg_ref, bt_ref, xgo_ref):
    xg = xg_ref[...]
    xgo_ref[...] = _leaky(
        jnp.dot(pooled_ref[...], wtp_ref[...],
                preferred_element_type=jnp.float32)
        + jnp.dot(xg, wtg_ref[...], preferred_element_type=jnp.float32)
        + bt_ref[...]) + xg


_tc_d = pl.pallas_call(
    _tc_d_body, out_shape=jax.ShapeDtypeStruct((G, EMB), jnp.float32))


def kernel(x, step_idx, edge_attr, edge_index, batch_ind, num_graphs,
           Wm, bm, Wa, ba, Wgm, bgm, Wgf, bgf, Wt, bt):
    src = edge_index[0].astype(jnp.int32)
    dst = edge_index[1].astype(jnp.int32)
    perm = jnp.argsort(dst)
    src_s = src[perm]
    dst_s = dst[perm]
    dst_f = dst_s.astype(jnp.float32)
    eap = edge_attr[perm]
    bounds = jnp.arange(33, dtype=jnp.int32) * NPW
    starts = jnp.searchsorted(dst_s, bounds).astype(jnp.int32)
    starts = jnp.pad(starts, (0, 15), constant_values=E)
    bi_c = batch_ind.astype(jnp.int32).reshape(N, 1)

    xg0 = jnp.zeros((G, EMB), jnp.float32)
    xm0 = _mm(x, Wm[0][:EMB])
    wm_next = jnp.roll(Wm, -1, axis=0)[:, :EMB]

    def step(carry, ws):
        xc, xgc, xmc = carry
        wmi, wai, bai, bmi, wgmi, bgmi, wgfi, bgfi, wti, bti, wmn = ws
        eam = _mm_ea(eap, wmi[EMB:])
        raw = _sc_agg(xmc, src_s, dst_f, eam, starts).reshape(NPAD, EMB)
        x2, xmn, feat, gl, gmax = _tc_a(
            bi_c, xc, raw, xgc,
            bmi.reshape(1, EMB),
            wai[:EMB], wai[EMB:2 * EMB], wai[2 * EMB:],
            bai.reshape(1, EMB),
            wgmi.reshape(1, EMB),
            bgmi.reshape(1, 1),
            wgfi, bgfi.reshape(1, EMB),
            wmn)
        gsum = _tc_b(bi_c, gl, gmax)
        pooled = _tc_c(bi_c, gl, feat, gmax, gsum)
        xg2 = _tc_d(pooled, xgc, wti[:EMB], wti[EMB:], bti.reshape(1, EMB))
        return (x2, xg2, xmn), None

    (xf, xgf, _), _ = lax.scan(
        step, (x, xg0, xm0),
        (Wm, Wa, ba, bm, Wgm, bgm, Wgf, bgf, Wt, bt, wm_next))
    return (xf, xgf)
